# Initial kernel scaffold; baseline (speedup 1.0000x reference)
import jax
import jax.numpy as jnp
from jax.experimental import pallas as pl


def kernel(x, edge_index, W1, b1, W2, b2):
    raise NotImplementedError("write your pallas kernel here")


# speedup vs baseline: 20.1072x; 20.1072x over previous
import functools

import jax
import jax.numpy as jnp
from jax import lax
from jax.experimental import pallas as pl
from jax.experimental.pallas import tpu as pltpu
from jax.experimental.pallas import tpu_sc as plsc

N = 10000
E = 320000
D = 128

_NC = 2
_NS = 16
_NW = _NC * _NS
_EPT = E // _NW
_C = 80
_NCHUNK = _EPT // _C
_RB = 80
_NRB = N // _RB
_RBI = (_NRB + _NS - 1) // _NS
_MM_B = 1000

_mesh = plsc.VectorSubcoreMesh(core_axis_name="c", subcore_axis_name="s")


@functools.partial(
    pl.kernel,
    out_type=jax.ShapeDtypeStruct((_NC * N, 16), jnp.float32),
    mesh=_mesh,
    scratch_types=[
        pltpu.VMEM((_NCHUNK, _C), jnp.int32),
        pltpu.VMEM((_C, 16), jnp.float32),
        pltpu.VMEM((_RB, 16), jnp.float32),
        pltpu.VMEM_SHARED((N, 16), jnp.float32),
    ],
)
def _sc_deg(dst_hbm, out_hbm, di_v, ones_v, zb_v, acc_sh):
    c = lax.axis_index("c")
    s = lax.axis_index("s")
    t = s * _NC + c

    @pl.loop(0, _C)
    def _(i):
        ones_v[i, pl.ds(0, 16)] = jnp.ones((16,), jnp.float32)

    @pl.loop(0, _RB)
    def _(i):
        zb_v[i, pl.ds(0, 16)] = jnp.zeros((16,), jnp.float32)

    @pl.loop(0, _RBI)
    def _(i):
        k = s + i * _NS

        @pl.when(k < _NRB)
        def _():
            pltpu.sync_copy(zb_v, acc_sh.at[pl.ds(k * _RB, _RB)])

    plsc.subcore_barrier()

    pltpu.sync_copy(dst_hbm.at[t], di_v)

    @pl.loop(0, _NCHUNK)
    def _(it):
        pltpu.sync_copy(ones_v, acc_sh.at[di_v.at[it]], add=True)

    plsc.subcore_barrier()

    @pl.loop(0, _RBI)
    def _(i):
        k = s + i * _NS

        @pl.when(k < _NRB)
        def _():
            pltpu.sync_copy(acc_sh.at[pl.ds(k * _RB, _RB)],
                            out_hbm.at[pl.ds(c * N + k * _RB, _RB)])


@functools.partial(
    pl.kernel,
    out_type=jax.ShapeDtypeStruct((_NC * N, D), jnp.float32),
    mesh=_mesh,
    scratch_types=[
        pltpu.VMEM((_NCHUNK, _C), jnp.int32),
        pltpu.VMEM((_NCHUNK, _C), jnp.int32),
        pltpu.VMEM((_C, D), jnp.float32),
        pltpu.VMEM_SHARED((N, D), jnp.float32),
        pltpu.SemaphoreType.DMA,
    ],
)
def _sc_agg(xs_hbm, src_hbm, dst_hbm, out_hbm, si_v, di_v, rows_v, acc_sh, sem):
    c = lax.axis_index("c")
    s = lax.axis_index("s")
    t = s * _NC + c

    @pl.loop(0, _RB)
    def _(i):
        @pl.loop(0, D, step=16)
        def _(j):
            rows_v[i, pl.ds(j, 16)] = jnp.zeros((16,), jnp.float32)

    @pl.loop(0, _RBI)
    def _(i):
        k = s + i * _NS

        @pl.when(k < _NRB)
        def _():
            pltpu.sync_copy(rows_v, acc_sh.at[pl.ds(k * _RB, _RB)])

    plsc.subcore_barrier()

    pltpu.sync_copy(src_hbm.at[t], si_v)
    pltpu.sync_copy(dst_hbm.at[t], di_v)

    @pl.loop(0, _NCHUNK)
    def _(it):
        pltpu.async_copy(xs_hbm.at[si_v.at[it]], rows_v, sem).wait()
        pltpu.sync_copy(rows_v, acc_sh.at[di_v.at[it]], add=True)

    plsc.subcore_barrier()

    @pl.loop(0, _RBI)
    def _(i):
        k = s + i * _NS

        @pl.when(k < _NRB)
        def _():
            pltpu.sync_copy(acc_sh.at[pl.ds(k * _RB, _RB)],
                            out_hbm.at[pl.ds(c * N + k * _RB, _RB)])


def _dis_block(dp_ref):
    deg = dp_ref[0, :, 0:1] + dp_ref[1, :, 0:1] + 1.0
    return lax.rsqrt(deg)


def _tc_mm(x, W):
    def body(x_ref, w_ref, o_ref):
        o_ref[...] = jnp.dot(x_ref[...], w_ref[...],
                             preferred_element_type=jnp.float32)

    return pl.pallas_call(
        body,
        grid=(N // _MM_B,),
        in_specs=[
            pl.BlockSpec((_MM_B, D), lambda i: (i, 0)),
            pl.BlockSpec((D, D), lambda i: (0, 0)),
        ],
        out_specs=pl.BlockSpec((_MM_B, D), lambda i: (i, 0)),
        out_shape=jax.ShapeDtypeStruct((N, D), jnp.float32),
    )(x, W)


def _tc_scale(degp, xw):
    def body(dp_ref, xw_ref, o_ref):
        o_ref[...] = xw_ref[...] * _dis_block(dp_ref)

    return pl.pallas_call(
        body,
        grid=(N // _MM_B,),
        in_specs=[
            pl.BlockSpec((_NC, _MM_B, 16), lambda i: (0, i, 0)),
            pl.BlockSpec((_MM_B, D), lambda i: (i, 0)),
        ],
        out_specs=pl.BlockSpec((_MM_B, D), lambda i: (i, 0)),
        out_shape=jax.ShapeDtypeStruct((N, D), jnp.float32),
    )(degp, xw)


def _tc_layer(degp, parts, xw, b, W2):
    def body(dp_ref, p_ref, xw_ref, b_ref, w2_ref, xw2_ref, xs2_ref):
        dis = _dis_block(dp_ref)
        h = dis * (p_ref[0] + p_ref[1]) + (dis * dis) * xw_ref[...] + b_ref[...]
        h = jnp.maximum(h, 0.0)
        xw2 = jnp.dot(h, w2_ref[...], preferred_element_type=jnp.float32)
        xw2_ref[...] = xw2
        xs2_ref[...] = xw2 * dis

    return pl.pallas_call(
        body,
        grid=(N // _MM_B,),
        in_specs=[
            pl.BlockSpec((_NC, _MM_B, 16), lambda i: (0, i, 0)),
            pl.BlockSpec((_NC, _MM_B, D), lambda i: (0, i, 0)),
            pl.BlockSpec((_MM_B, D), lambda i: (i, 0)),
            pl.BlockSpec((1, D), lambda i: (0, 0)),
            pl.BlockSpec((D, D), lambda i: (0, 0)),
        ],
        out_specs=[
            pl.BlockSpec((_MM_B, D), lambda i: (i, 0)),
            pl.BlockSpec((_MM_B, D), lambda i: (i, 0)),
        ],
        out_shape=[
            jax.ShapeDtypeStruct((N, D), jnp.float32),
            jax.ShapeDtypeStruct((N, D), jnp.float32),
        ],
    )(degp, parts, xw, b, W2)


def _tc_out(degp, parts, xw, b):
    def body(dp_ref, p_ref, xw_ref, b_ref, o_ref):
        dis = _dis_block(dp_ref)
        o_ref[...] = (dis * (p_ref[0] + p_ref[1])
                      + (dis * dis) * xw_ref[...] + b_ref[...])

    return pl.pallas_call(
        body,
        grid=(N // _MM_B,),
        in_specs=[
            pl.BlockSpec((_NC, _MM_B, 16), lambda i: (0, i, 0)),
            pl.BlockSpec((_NC, _MM_B, D), lambda i: (0, i, 0)),
            pl.BlockSpec((_MM_B, D), lambda i: (i, 0)),
            pl.BlockSpec((1, D), lambda i: (0, 0)),
        ],
        out_specs=pl.BlockSpec((_MM_B, D), lambda i: (i, 0)),
        out_shape=jax.ShapeDtypeStruct((N, D), jnp.float32),
    )(degp, parts, xw, b)


def kernel(x, edge_index, W1, b1, W2, b2):
    src = edge_index[0].reshape(_NW, _NCHUNK, _C)
    dst = edge_index[1].reshape(_NW, _NCHUNK, _C)

    degp = _sc_deg(dst).reshape(_NC, N, 16)
    xw1 = _tc_mm(x, W1)
    xs1 = _tc_scale(degp, xw1)
    parts1 = _sc_agg(xs1, src, dst).reshape(_NC, N, D)
    xw2, xs2 = _tc_layer(degp, parts1, xw1, b1.reshape(1, D), W2)
    parts2 = _sc_agg(xs2, src, dst).reshape(_NC, N, D)
    return _tc_out(degp, parts2, xw2, b2.reshape(1, D))


# speedup vs baseline: 25.1388x; 1.2502x over previous
import functools

import jax
import jax.numpy as jnp
from jax import lax
from jax.experimental import pallas as pl
from jax.experimental.pallas import tpu as pltpu
from jax.experimental.pallas import tpu_sc as plsc

N = 10000
E = 320000
D = 128

_NC = 2
_NS = 16
_NW = _NC * _NS
_EPT = E // _NW
_C = 80
_NCHUNK = _EPT // _C
_RB = 80
_NRB = N // _RB
_RBI = (_NRB + _NS - 1) // _NS
_MM_B = 1000

_mesh = plsc.VectorSubcoreMesh(core_axis_name="c", subcore_axis_name="s")


@functools.partial(
    pl.kernel,
    out_type=jax.ShapeDtypeStruct((_NC * N, 16), jnp.float32),
    mesh=_mesh,
    scratch_types=[
        pltpu.VMEM((_NCHUNK, _C), jnp.int32),
        pltpu.VMEM((_C, 16), jnp.float32),
        pltpu.VMEM((_RB, 16), jnp.float32),
        pltpu.VMEM_SHARED((N, 16), jnp.float32),
    ],
)
def _sc_deg(dst_hbm, out_hbm, di_v, ones_v, zb_v, acc_sh):
    c = lax.axis_index("c")
    s = lax.axis_index("s")
    t = s * _NC + c

    @pl.loop(0, _C)
    def _(i):
        ones_v[i, pl.ds(0, 16)] = jnp.ones((16,), jnp.float32)

    @pl.loop(0, _RB)
    def _(i):
        zb_v[i, pl.ds(0, 16)] = jnp.zeros((16,), jnp.float32)

    @pl.loop(0, _RBI)
    def _(i):
        k = s + i * _NS

        @pl.when(k < _NRB)
        def _():
            pltpu.sync_copy(zb_v, acc_sh.at[pl.ds(k * _RB, _RB)])

    plsc.subcore_barrier()

    pltpu.sync_copy(dst_hbm.at[t], di_v)

    @pl.loop(0, _NCHUNK)
    def _(it):
        pltpu.sync_copy(ones_v, acc_sh.at[di_v.at[it]], add=True)

    plsc.subcore_barrier()

    @pl.loop(0, _RBI)
    def _(i):
        k = s + i * _NS

        @pl.when(k < _NRB)
        def _():
            pltpu.sync_copy(acc_sh.at[pl.ds(k * _RB, _RB)],
                            out_hbm.at[pl.ds(c * N + k * _RB, _RB)])


@functools.partial(
    pl.kernel,
    out_type=jax.ShapeDtypeStruct((_NC * N, D), jnp.float32),
    mesh=_mesh,
    scratch_types=[
        pltpu.VMEM((_EPT,), jnp.int32),
        pltpu.VMEM((_NCHUNK, _C), jnp.int32),
        pltpu.VMEM((_C, D), jnp.float32),
        pltpu.VMEM((_C, D), jnp.float32),
        pltpu.VMEM_SHARED((N, D), jnp.float32),
        pltpu.SemaphoreType.DMA,
        pltpu.SemaphoreType.DMA,
    ],
)
def _sc_agg(xs_hbm, src_hbm, dst_hbm, out_hbm, si_v, di_v, r0_v, r1_v,
            acc_sh, sem0, sem1):
    c = lax.axis_index("c")
    s = lax.axis_index("s")
    t = s * _NC + c

    pltpu.sync_copy(src_hbm.at[t], si_v)
    pltpu.sync_copy(dst_hbm.at[t], di_v)

    @pl.loop(0, _RB)
    def _(i):
        @pl.loop(0, D, step=16)
        def _(j):
            r0_v[i, pl.ds(j, 16)] = jnp.zeros((16,), jnp.float32)

    @pl.loop(0, _RBI)
    def _(i):
        k = s + i * _NS

        @pl.when(k < _NRB)
        def _():
            pltpu.sync_copy(r0_v, acc_sh.at[pl.ds(k * _RB, _RB)])

    def _start(it, buf, sem):
        pltpu.async_copy(xs_hbm.at[si_v.at[pl.ds(it * _C, _C)]], buf, sem)

    def _finish(it, buf, sem):
        pltpu.make_async_copy(xs_hbm.at[si_v.at[pl.ds(it * _C, _C)]], buf,
                              sem).wait()
        pltpu.sync_copy(buf, acc_sh.at[di_v.at[it]], add=True)

    plsc.subcore_barrier()

    def _wait(it, buf, sem):
        pltpu.make_async_copy(xs_hbm.at[si_v.at[pl.ds(it * _C, _C)]], buf,
                              sem).wait()

    def _scat(it, buf):
        pltpu.sync_copy(buf, acc_sh.at[di_v.at[it]], add=True)

    _start(0, r0_v, sem0)

    @pl.loop(0, _NCHUNK // 2)
    def _(p):
        i = p * 2
        _wait(i, r0_v, sem0)
        _start(i + 1, r1_v, sem1)
        _scat(i, r0_v)
        _wait(i + 1, r1_v, sem1)

        @pl.when(i + 2 < _NCHUNK)
        def _():
            _start(i + 2, r0_v, sem0)

        _scat(i + 1, r1_v)

    if _NCHUNK % 2:
        _wait(_NCHUNK - 1, r0_v, sem0)
        _scat(_NCHUNK - 1, r0_v)

    plsc.subcore_barrier()

    @pl.loop(0, _RBI)
    def _(i):
        k = s + i * _NS

        @pl.when(k < _NRB)
        def _():
            pltpu.sync_copy(acc_sh.at[pl.ds(k * _RB, _RB)],
                            out_hbm.at[pl.ds(c * N + k * _RB, _RB)])


def _dis_block(dp_ref):
    deg = dp_ref[0, :, 0:1] + dp_ref[1, :, 0:1] + 1.0
    return lax.rsqrt(deg)


def _tc_mm(x, W):
    def body(x_ref, w_ref, o_ref):
        o_ref[...] = jnp.dot(x_ref[...], w_ref[...],
                             preferred_element_type=jnp.float32)

    return pl.pallas_call(
        body,
        grid=(N // _MM_B,),
        in_specs=[
            pl.BlockSpec((_MM_B, D), lambda i: (i, 0)),
            pl.BlockSpec((D, D), lambda i: (0, 0)),
        ],
        out_specs=pl.BlockSpec((_MM_B, D), lambda i: (i, 0)),
        out_shape=jax.ShapeDtypeStruct((N, D), jnp.float32),
    )(x, W)


def _tc_scale(degp, xw):
    def body(dp_ref, xw_ref, o_ref):
        o_ref[...] = xw_ref[...] * _dis_block(dp_ref)

    return pl.pallas_call(
        body,
        grid=(N // _MM_B,),
        in_specs=[
            pl.BlockSpec((_NC, _MM_B, 16), lambda i: (0, i, 0)),
            pl.BlockSpec((_MM_B, D), lambda i: (i, 0)),
        ],
        out_specs=pl.BlockSpec((_MM_B, D), lambda i: (i, 0)),
        out_shape=jax.ShapeDtypeStruct((N, D), jnp.float32),
    )(degp, xw)


def _tc_layer(degp, parts, xw, b, W2):
    def body(dp_ref, p_ref, xw_ref, b_ref, w2_ref, xw2_ref, xs2_ref):
        dis = _dis_block(dp_ref)
        h = dis * (p_ref[0] + p_ref[1]) + (dis * dis) * xw_ref[...] + b_ref[...]
        h = jnp.maximum(h, 0.0)
        xw2 = jnp.dot(h, w2_ref[...], preferred_element_type=jnp.float32)
        xw2_ref[...] = xw2
        xs2_ref[...] = xw2 * dis

    return pl.pallas_call(
        body,
        grid=(N // _MM_B,),
        in_specs=[
            pl.BlockSpec((_NC, _MM_B, 16), lambda i: (0, i, 0)),
            pl.BlockSpec((_NC, _MM_B, D), lambda i: (0, i, 0)),
            pl.BlockSpec((_MM_B, D), lambda i: (i, 0)),
            pl.BlockSpec((1, D), lambda i: (0, 0)),
            pl.BlockSpec((D, D), lambda i: (0, 0)),
        ],
        out_specs=[
            pl.BlockSpec((_MM_B, D), lambda i: (i, 0)),
            pl.BlockSpec((_MM_B, D), lambda i: (i, 0)),
        ],
        out_shape=[
            jax.ShapeDtypeStruct((N, D), jnp.float32),
            jax.ShapeDtypeStruct((N, D), jnp.float32),
        ],
    )(degp, parts, xw, b, W2)


def _tc_out(degp, parts, xw, b):
    def body(dp_ref, p_ref, xw_ref, b_ref, o_ref):
        dis = _dis_block(dp_ref)
        o_ref[...] = (dis * (p_ref[0] + p_ref[1])
                      + (dis * dis) * xw_ref[...] + b_ref[...])

    return pl.pallas_call(
        body,
        grid=(N // _MM_B,),
        in_specs=[
            pl.BlockSpec((_NC, _MM_B, 16), lambda i: (0, i, 0)),
            pl.BlockSpec((_NC, _MM_B, D), lambda i: (0, i, 0)),
            pl.BlockSpec((_MM_B, D), lambda i: (i, 0)),
            pl.BlockSpec((1, D), lambda i: (0, 0)),
        ],
        out_specs=pl.BlockSpec((_MM_B, D), lambda i: (i, 0)),
        out_shape=jax.ShapeDtypeStruct((N, D), jnp.float32),
    )(degp, parts, xw, b)


def kernel(x, edge_index, W1, b1, W2, b2):
    src_a = edge_index[0].reshape(_NW, _EPT)
    dst_a = edge_index[1].reshape(_NW, _NCHUNK, _C)

    degp = _sc_deg(dst_a).reshape(_NC, N, 16)
    xw1 = _tc_mm(x, W1)
    xs1 = _tc_scale(degp, xw1)
    parts1 = _sc_agg(xs1, src_a, dst_a).reshape(_NC, N, D)
    xw2, xs2 = _tc_layer(degp, parts1, xw1, b1.reshape(1, D), W2)
    parts2 = _sc_agg(xs2, src_a, dst_a).reshape(_NC, N, D)
    return _tc_out(degp, parts2, xw2, b2.reshape(1, D))
